# baseline (speedup 1.0000x reference)
import functools

import jax
import jax.numpy as jnp
from jax import lax
from jax.experimental import pallas as pl
from jax.experimental.pallas import tpu as pltpu
from jax.experimental.pallas import tpu_sc as plsc

B = 16384
NUM_EMB_ROWS = 2000000
ED = 16
TOT = 2 * B
NC, NS = 2, 16
NW = NC * NS
PER_W = TOT // NW
CH = 128
NCH = PER_W // CH


def _gather_body(idx_hbm, table_hbm, out_hbm, idx_v, rows_v, sem):
    wid = lax.axis_index("s") * NC + lax.axis_index("c")
    base = wid * PER_W
    pltpu.sync_copy(idx_hbm.at[wid], idx_v)
    copies = []
    for j in range(NCH):
        copies.append(
            pltpu.async_copy(
                table_hbm.at[idx_v.at[j]],
                rows_v.at[pl.ds(j * CH, CH)],
                sem,
            )
        )
    for c in copies:
        c.wait()
    pltpu.sync_copy(rows_v, out_hbm.at[pl.ds(base, PER_W)])


_gather = functools.partial(
    pl.kernel,
    out_type=jax.ShapeDtypeStruct((TOT, ED), jnp.float32),
    mesh=plsc.VectorSubcoreMesh(core_axis_name="c", subcore_axis_name="s"),
    scratch_types=[
        pltpu.VMEM((NCH, CH), jnp.int32),
        pltpu.VMEM((PER_W, ED), jnp.float32),
        pltpu.SemaphoreType.DMA,
    ],
    compiler_params=pltpu.CompilerParams(use_tc_tiling_on_sc=False),
)(_gather_body)


def _mlp_body(u_ref, v_ref, W1_ref, b1_ref, W2_ref, b2_ref, W3_ref, b3_ref,
              Wfc_ref, bfc_ref, out_ref):
    u = u_ref[...]
    v = v_ref[...]
    h = jnp.concatenate([u, v], axis=1)
    h = jnp.maximum(jnp.dot(h, W1_ref[...], preferred_element_type=jnp.float32)
                    + b1_ref[...], 0.0)
    h = jnp.maximum(jnp.dot(h, W2_ref[...], preferred_element_type=jnp.float32)
                    + b2_ref[...], 0.0)
    h = jnp.maximum(jnp.dot(h, W3_ref[...], preferred_element_type=jnp.float32)
                    + b3_ref[...], 0.0)
    gmf = u * v
    c = jnp.concatenate([gmf, h], axis=1)
    out_ref[...] = jnp.dot(c, Wfc_ref[...], preferred_element_type=jnp.float32) + bfc_ref[...]


def kernel(x, table, W1, b1, W2, b2, W3, b3, Wfc, bfc):
    offsets = jnp.array([0, NUM_EMB_ROWS // 2], dtype=x.dtype)
    idx = (x + offsets[None, :]).astype(jnp.int32)
    idx_cm = idx.T.reshape(NW, NCH, CH)

    rows = _gather(idx_cm, table)
    u = rows[:B]
    v = rows[B:]

    out = pl.pallas_call(
        _mlp_body,
        out_shape=jax.ShapeDtypeStruct((B, 1), jnp.float32),
    )(u, v, W1, b1.reshape(1, -1), W2, b2.reshape(1, -1),
      W3, b3.reshape(1, -1), Wfc, bfc.reshape(1, 1))
    return out[:, 0]
